# baseline (speedup 1.0000x reference)
import functools

import jax
import jax.numpy as jnp
from jax import lax
from jax.experimental import pallas as pl
from jax.experimental.pallas import tpu as pltpu
from jax.experimental.pallas import tpu_sc as plsc

B = 16384
G = 20
D = 64
NC = 2
NS = 16
NW = NC * NS
BPW = B // NW
WC = 32
NCHUNK = BPW // WC
IPC = WC * G
IB = 128
NGATH = IPC // IB
LG = D // 16


def _sc_body(center_hbm, ngram_hbm, win_hbm, wsub_hbm, out_hbm,
             cidx_v, nidx_v, crow_v, srow_v, orow_v, sem):
    wid = lax.axis_index("s") * NC + lax.axis_index("c")
    base = wid * BPW

    def chunk_body(ci, carry):
        wbase = base + ci * WC
        pltpu.sync_copy(center_hbm.at[pl.ds(wbase, WC)], cidx_v)
        pltpu.sync_copy(ngram_hbm.at[pl.ds(wbase * G, IPC)], nidx_v)

        cps = [pltpu.async_copy(win_hbm.at[cidx_v], crow_v, sem)]
        for j in range(NGATH):
            cps.append(pltpu.async_copy(
                wsub_hbm.at[nidx_v.at[pl.ds(j * IB, IB)]],
                srow_v.at[pl.ds(j * IB, IB)], sem))
        for cp in cps:
            cp.wait()

        def word_body(w, c):
            accs = [crow_v[w, pl.ds(16 * l, 16)] for l in range(LG)]
            for g in range(G):
                r = w * G + g
                for l in range(LG):
                    accs[l] = accs[l] + srow_v[r, pl.ds(16 * l, 16)]
            for l in range(LG):
                orow_v[w, pl.ds(16 * l, 16)] = accs[l]
            return c

        lax.fori_loop(0, WC, word_body, 0)
        pltpu.sync_copy(orow_v, out_hbm.at[pl.ds(wbase, WC)])
        return carry

    lax.fori_loop(0, NCHUNK, chunk_body, 0)


_mesh = plsc.VectorSubcoreMesh(core_axis_name="c", subcore_axis_name="s")

_sc_call = functools.partial(
    pl.kernel,
    mesh=_mesh,
    out_type=jax.ShapeDtypeStruct((B, D), jnp.float32),
    scratch_types=[
        pltpu.VMEM((WC,), jnp.int32),
        pltpu.VMEM((IPC,), jnp.int32),
        pltpu.VMEM((WC, D), jnp.float32),
        pltpu.VMEM((IPC, D), jnp.float32),
        pltpu.VMEM((WC, D), jnp.float32),
        pltpu.SemaphoreType.DMA,
    ],
    compiler_params=pltpu.CompilerParams(use_tc_tiling_on_sc=False),
)(_sc_body)


def kernel(center_ids, ngram_ids, W_in, W_sub):
    center = center_ids.astype(jnp.int32)
    ngram_flat = ngram_ids.astype(jnp.int32).reshape(B * G)
    return _sc_call(center, ngram_flat, W_in, W_sub)
